# Initial kernel scaffold; baseline (speedup 1.0000x reference)
import jax
import jax.numpy as jnp
from jax.experimental import pallas as pl


def kernel(lights, settlement, admin_ids, census_totals):
    raise NotImplementedError("write your pallas kernel here")


# speedup vs baseline: 148.5309x; 148.5309x over previous
import functools

import jax
import jax.numpy as jnp
from jax import lax
from jax.experimental import pallas as pl
from jax.experimental.pallas import tpu as pltpu
from jax.experimental.pallas import tpu_sc as plsc

LAMBDA_L = 0.01
LAMBDA_S = 0.01
EPS = 1e-08

B, H, W = 8, 512, 512
NA = 64
N = B * H * W
NC, NS, L = 2, 16, 16
NW = NC * NS
PER_TILE = N // NW
CHUNK = 16384
NCHUNK = PER_TILE // CHUNK
TILES_PER_BATCH = NW // B

_mesh = plsc.VectorSubcoreMesh(core_axis_name="c", subcore_axis_name="s")
_params = pltpu.CompilerParams(needs_layout_passes=False)


@functools.partial(
    pl.kernel,
    mesh=_mesh,
    compiler_params=_params,
    out_type=jax.ShapeDtypeStruct((NW * NA,), jnp.float32),
    scratch_types=[
        pltpu.VMEM((CHUNK,), jnp.float32),
        pltpu.VMEM((CHUNK,), jnp.float32),
        pltpu.VMEM((CHUNK,), jnp.int32),
        pltpu.VMEM((L * NA,), jnp.float32),
        pltpu.VMEM((NA,), jnp.float32),
    ],
)
def _phase1(l_hbm, s_hbm, a_hbm, part_hbm, lbuf, sbuf, abuf, accum, sums):
    wid = lax.axis_index("c") * NS + lax.axis_index("s")
    base = wid * PER_TILE
    zero = jnp.zeros((L,), jnp.float32)
    for k in range(NA):
        accum[pl.ds(k * L, L)] = zero
    lane_off = jnp.arange(L, dtype=jnp.int32) * NA

    def chunk_body(ci, _):
        off = base + ci * CHUNK
        pltpu.sync_copy(l_hbm.at[pl.ds(off, CHUNK)], lbuf)
        pltpu.sync_copy(s_hbm.at[pl.ds(off, CHUNK)], sbuf)
        pltpu.sync_copy(a_hbm.at[pl.ds(off, CHUNK)], abuf)

        def body(i, _):
            p = pl.ds(i * L, L)
            score = (lbuf[p] + LAMBDA_L) * (sbuf[p] + LAMBDA_S)
            plsc.addupdate_scatter(accum, [lane_off + abuf[p]], score)
            return 0

        lax.fori_loop(0, CHUNK // L, body, 0)
        return 0

    lax.fori_loop(0, NCHUNK, chunk_body, 0)

    for k in range(NA // L):
        t = accum[pl.ds(k * L, L)]
        for lane in range(1, L):
            t = t + accum[pl.ds(lane * NA + k * L, L)]
        sums[pl.ds(k * L, L)] = t
    pltpu.sync_copy(sums, part_hbm.at[pl.ds(wid * NA, NA)])


@functools.partial(
    pl.kernel,
    mesh=_mesh,
    compiler_params=_params,
    out_type=jax.ShapeDtypeStruct((N,), jnp.float32),
    scratch_types=[
        pltpu.VMEM((CHUNK,), jnp.float32),
        pltpu.VMEM((CHUNK,), jnp.float32),
        pltpu.VMEM((CHUNK,), jnp.int32),
        pltpu.VMEM((CHUNK,), jnp.float32),
        pltpu.VMEM((TILES_PER_BATCH * NA,), jnp.float32),
        pltpu.VMEM((NA,), jnp.float32),
        pltpu.VMEM((NA,), jnp.float32),
    ],
)
def _phase2(l_hbm, s_hbm, a_hbm, part_hbm, c_hbm, out_hbm,
            lbuf, sbuf, abuf, obuf, pbuf, cbuf, fbuf):
    wid = lax.axis_index("c") * NS + lax.axis_index("s")
    batch = wid // TILES_PER_BATCH
    pltpu.sync_copy(part_hbm.at[pl.ds(batch * TILES_PER_BATCH * NA,
                                      TILES_PER_BATCH * NA)], pbuf)
    pltpu.sync_copy(c_hbm, cbuf)
    for k in range(NA // L):
        t = pbuf[pl.ds(k * L, L)]
        for j in range(1, TILES_PER_BATCH):
            t = t + pbuf[pl.ds(j * NA + k * L, L)]
        fbuf[pl.ds(k * L, L)] = cbuf[pl.ds(k * L, L)] / (t + EPS)

    base = wid * PER_TILE

    def chunk_body(ci, _):
        off = base + ci * CHUNK
        pltpu.sync_copy(l_hbm.at[pl.ds(off, CHUNK)], lbuf)
        pltpu.sync_copy(s_hbm.at[pl.ds(off, CHUNK)], sbuf)
        pltpu.sync_copy(a_hbm.at[pl.ds(off, CHUNK)], abuf)

        def body(i, _):
            p = pl.ds(i * L, L)
            score = (lbuf[p] + LAMBDA_L) * (sbuf[p] + LAMBDA_S)
            f = plsc.load_gather(fbuf, [abuf[p]])
            obuf[p] = score * f
            return 0

        lax.fori_loop(0, CHUNK // L, body, 0)
        pltpu.sync_copy(obuf, out_hbm.at[pl.ds(off, CHUNK)])
        return 0

    lax.fori_loop(0, NCHUNK, chunk_body, 0)


def kernel(lights, settlement, admin_ids, census_totals):
    l_flat = lights.reshape(-1)
    s_flat = settlement.reshape(-1)
    a_flat = admin_ids.reshape(-1)
    partials = _phase1(l_flat, s_flat, a_flat)
    out = _phase2(l_flat, s_flat, a_flat, partials, census_totals)
    return out.reshape(lights.shape)
